# Initial kernel scaffold; baseline (speedup 1.0000x reference)
import jax
import jax.numpy as jnp
from jax import lax
from jax.experimental import pallas as pl

_NUM_CLUSTERS = 512
_MAX_ITERS = 10
_TOL = 1e-4
_TILE = 2048


def _kmeans_body(data_ref, cents_ref, labels_ref):
    n, d = data_ref.shape
    k = cents_ref.shape[0]
    num_tiles = n // _TILE

    def iter_body(_, carry):
        cents, done = carry
        c2 = jnp.sum(cents * cents, axis=1)[None, :]

        def tile_body(t, acc):
            sums, counts = acc
            x = data_ref[pl.ds(t * _TILE, _TILE), :]
            x2 = jnp.sum(x * x, axis=1, keepdims=True)
            dots = lax.dot_general(
                x, cents,
                dimension_numbers=(((1,), (1,)), ((), ())),
                preferred_element_type=jnp.float32,
            )
            d2 = jnp.maximum(x2 + c2 - 2.0 * dots, 0.0)
            labels = jnp.argmin(d2, axis=1).astype(jnp.int32)
            labels_ref[pl.ds(t * (_TILE // 128), _TILE // 128), :] = (
                labels.reshape(_TILE // 128, 128))
            oh = (lax.broadcasted_iota(jnp.int32, (k, _TILE), 0)
                  == labels[None, :]).astype(jnp.float32)
            sums = sums + lax.dot_general(
                oh, x,
                dimension_numbers=(((1,), (0,)), ((), ())),
                preferred_element_type=jnp.float32,
            )
            counts = counts + lax.dot_general(
                oh, jnp.ones((_TILE, 1), jnp.float32),
                dimension_numbers=(((1,), (0,)), ((), ())),
                preferred_element_type=jnp.float32,
            )
            return sums, counts

        sums0 = jnp.zeros((k, d), jnp.float32)
        counts0 = jnp.zeros((k, 1), jnp.float32)
        sums, counts = lax.fori_loop(0, num_tiles, tile_body, (sums0, counts0))

        new_cents = jnp.where(
            counts > 0.0, sums / jnp.maximum(counts, 1.0), cents)
        diff = new_cents - cents
        converged = jnp.sum(diff * diff) < _TOL * _TOL
        cents = jnp.where(done | converged, cents, new_cents)
        return cents, done | converged

    cents0 = cents_ref[...]
    done0 = jnp.asarray(False)
    lax.fori_loop(0, _MAX_ITERS, iter_body, (cents0, done0))


@jax.jit
def kernel(data):
    n = data.shape[0]
    perm = jax.random.permutation(jax.random.key(1), n)
    cents0 = jnp.take(data, perm[:_NUM_CLUSTERS], axis=0)
    labels2d = pl.pallas_call(
        _kmeans_body,
        out_shape=jax.ShapeDtypeStruct((n // 128, 128), jnp.int32),
    )(data, cents0)
    return labels2d.reshape(n)


# speedup vs baseline: 1.6866x; 1.6866x over previous
import jax
import jax.numpy as jnp
from jax import lax
from jax.experimental import pallas as pl

_NUM_CLUSTERS = 512
_MAX_ITERS = 10
_TOL = 1e-4
_TILE = 2048


def _kmeans_body(data_ref, cents_ref, labels_ref):
    n, d = data_ref.shape
    k = cents_ref.shape[0]
    num_tiles = n // _TILE

    def iter_body(_, carry):
        cents, done = carry
        c2 = jnp.sum(cents * cents, axis=1)[None, :]

        def tile_body(t, acc):
            sums, counts = acc
            x = data_ref[pl.ds(t * _TILE, _TILE), :]
            x2 = jnp.sum(x * x, axis=1, keepdims=True)
            dots = lax.dot_general(
                x, cents,
                dimension_numbers=(((1,), (1,)), ((), ())),
                preferred_element_type=jnp.float32,
            )
            dist = jnp.sqrt(jnp.maximum(x2 + c2 - 2.0 * dots, 0.0))
            labels = jnp.argmin(dist, axis=1).astype(jnp.int32)
            labels_ref[pl.ds(t * (_TILE // 128), _TILE // 128), :] = (
                labels.reshape(_TILE // 128, 128))
            oh = (lax.broadcasted_iota(jnp.int32, (k, _TILE), 0)
                  == labels[None, :]).astype(jnp.float32)
            sums = sums + lax.dot_general(
                oh, x,
                dimension_numbers=(((1,), (0,)), ((), ())),
                preferred_element_type=jnp.float32,
                precision=lax.Precision.HIGHEST,
            )
            counts = counts + lax.dot_general(
                oh, jnp.ones((_TILE, 1), jnp.float32),
                dimension_numbers=(((1,), (0,)), ((), ())),
                preferred_element_type=jnp.float32,
            )
            return sums, counts

        sums0 = jnp.zeros((k, d), jnp.float32)
        counts0 = jnp.zeros((k, 1), jnp.float32)
        sums, counts = lax.fori_loop(0, num_tiles, tile_body, (sums0, counts0))

        new_cents = jnp.where(
            counts > 0.0, sums / jnp.maximum(counts, 1.0), cents)
        diff = new_cents - cents
        converged = jnp.sum(diff * diff) < _TOL * _TOL
        cents = jnp.where(done | converged, cents, new_cents)
        return cents, done | converged

    cents0 = cents_ref[...]
    done0 = jnp.asarray(False)
    lax.fori_loop(0, _MAX_ITERS, iter_body, (cents0, done0))


@jax.jit
def kernel(data):
    n = data.shape[0]
    perm = jax.random.permutation(jax.random.key(1), n)
    cents0 = jnp.take(data, perm[:_NUM_CLUSTERS], axis=0)
    labels2d = pl.pallas_call(
        _kmeans_body,
        out_shape=jax.ShapeDtypeStruct((n // 128, 128), jnp.int32),
    )(data, cents0)
    return labels2d.reshape(n)


# speedup vs baseline: 3.6170x; 2.1445x over previous
import jax
import jax.numpy as jnp
from jax import lax
from jax.experimental import pallas as pl
from jax.experimental.pallas import tpu as pltpu

_NUM_CLUSTERS = 512
_MAX_ITERS = 10
_TOL = 1e-4
_TILE = 4096


def _kmeans_body(data_ref, cents_ref, labels_ref, split_ref):
    n, d = data_ref.shape
    k = cents_ref.shape[0]
    num_tiles = n // _TILE
    rows128 = _TILE // 128

    def split_tile(t, _):
        x = data_ref[pl.ds(t * _TILE, _TILE), :]
        hi = x.astype(jnp.bfloat16).astype(jnp.float32)
        r1 = x - hi
        mid = r1.astype(jnp.bfloat16).astype(jnp.float32)
        lo = r1 - mid
        base = t * _TILE
        split_ref[pl.ds(base, _TILE), :d] = hi
        split_ref[pl.ds(base, _TILE), d:d + 1] = jnp.ones((_TILE, 1), jnp.float32)
        split_ref[pl.ds(n + base, _TILE), :d] = mid
        split_ref[pl.ds(n + base, _TILE), d:d + 1] = jnp.zeros((_TILE, 1), jnp.float32)
        split_ref[pl.ds(2 * n + base, _TILE), :d] = lo
        split_ref[pl.ds(2 * n + base, _TILE), d:d + 1] = jnp.zeros((_TILE, 1), jnp.float32)
        return 0

    lax.fori_loop(0, num_tiles, split_tile, 0)

    def iter_body(_, carry):
        cents, done = carry
        c2 = jnp.sum(cents * cents, axis=1)[None, :]

        def tile_body(t, acc):
            sums_aug = acc
            x = data_ref[pl.ds(t * _TILE, _TILE), :]
            x2 = jnp.sum(x * x, axis=1, keepdims=True)
            dots = lax.dot_general(
                x, cents,
                dimension_numbers=(((1,), (1,)), ((), ())),
                preferred_element_type=jnp.float32,
            )
            d2 = jnp.maximum(x2 + c2 - 2.0 * dots, 0.0)
            labels = jnp.argmin(d2, axis=1).astype(jnp.int32)
            labels_ref[pl.ds(t * rows128, rows128), :] = (
                labels.reshape(rows128, 128))
            oh = (lax.broadcasted_iota(jnp.int32, (k, _TILE), 0)
                  == labels[None, :]).astype(jnp.float32)
            base = t * _TILE
            for part in range(3):
                sums_aug = sums_aug + lax.dot_general(
                    oh, split_ref[pl.ds(part * n + base, _TILE), :],
                    dimension_numbers=(((1,), (0,)), ((), ())),
                    preferred_element_type=jnp.float32,
                )
            return sums_aug

        sums_aug = lax.fori_loop(
            0, num_tiles, tile_body, jnp.zeros((k, d + 1), jnp.float32))
        sums = sums_aug[:, :d]
        counts = sums_aug[:, d:d + 1]

        new_cents = jnp.where(
            counts > 0.0, sums / jnp.maximum(counts, 1.0), cents)
        diff = new_cents - cents
        converged = jnp.sum(diff * diff) < _TOL * _TOL
        cents = jnp.where(done | converged, cents, new_cents)
        return cents, done | converged

    cents0 = cents_ref[...]
    done0 = jnp.asarray(False)
    lax.fori_loop(0, _MAX_ITERS, iter_body, (cents0, done0))


@jax.jit
def kernel(data):
    n, d = data.shape
    perm = jax.random.permutation(jax.random.key(1), n)
    cents0 = jnp.take(data, perm[:_NUM_CLUSTERS], axis=0)
    labels2d = pl.pallas_call(
        _kmeans_body,
        out_shape=jax.ShapeDtypeStruct((n // 128, 128), jnp.int32),
        scratch_shapes=[pltpu.VMEM((3 * n, d + 1), jnp.float32)],
    )(data, cents0)
    return labels2d.reshape(n)
